# Initial kernel scaffold; baseline (speedup 1.0000x reference)
import jax
import jax.numpy as jnp
from jax.experimental import pallas as pl


def kernel(x, pos, edge_index, edge_attr, Wn1, bn1, Wn2, bn2, We1, be1, We2, be2, Wm1, bm1, Wm2, bm2, Wout, bout):
    raise NotImplementedError("write your pallas kernel here")


# speedup vs baseline: 2.1103x; 2.1103x over previous
import functools

import jax
import jax.numpy as jnp
import numpy as np
from jax import lax
from jax.experimental import pallas as pl
from jax.experimental.pallas import tpu as pltpu
from jax.experimental.pallas import tpu_sc as plsc

_N = 10000
_E = 160000
_NS = 24
_DP = 32
_PW = 16
_NC = 2
_NT = 16
_NW = _NC * _NT
_EPW = _E // _NW
_C = 1000
_NP = 10240
_RPT = _NP // _NT

_BE = 2000
_BN = 2000


def _mesh():
    return plsc.VectorSubcoreMesh(core_axis_name="c", subcore_axis_name="s")


@functools.lru_cache(maxsize=None)
def _make_sc_pos_deg():
  kern = functools.partial(
      pl.kernel, mesh=_mesh(),
      compiler_params=pltpu.CompilerParams(use_tc_tiling_on_sc=False),
      out_type=(
          jax.ShapeDtypeStruct((_E, _PW), jnp.float32),
          jax.ShapeDtypeStruct((_E, _PW), jnp.float32),
          jax.ShapeDtypeStruct((2 * _NP, _PW), jnp.float32),
      ),
      scratch_types=[
          pltpu.VMEM((_C,), jnp.int32),
          pltpu.VMEM((_C, _PW), jnp.float32),
          pltpu.VMEM((_C, _PW), jnp.float32),
          pltpu.VMEM_SHARED((_NP, _PW), jnp.float32),
          pltpu.SemaphoreType.DMA,
      ],
  )

  @kern
  def body(pos_h, src_h, dst_h, zero_h, ps_h, pd_h, deg_h,
           idx_v, rows_v, ones_v, acc_sh, sem):
    c = lax.axis_index("c")
    s = lax.axis_index("s")
    wid = s * _NC + c
    base = wid * _EPW

    def fill_ones(i, carry):
      ones_v[i] = jnp.ones((_PW,), jnp.float32)
      return carry
    lax.fori_loop(0, _C, fill_ones, 0)
    pltpu.sync_copy(zero_h, acc_sh.at[pl.ds(s * _RPT, _RPT)])
    plsc.subcore_barrier()

    def step_src(j, carry):
      off = pl.multiple_of(base + j * _C, 8)
      pltpu.sync_copy(src_h.at[pl.ds(off, _C)], idx_v)
      pltpu.async_copy(pos_h.at[idx_v], rows_v, sem).wait()
      pltpu.sync_copy(rows_v, ps_h.at[pl.ds(off, _C)])
      return carry
    lax.fori_loop(0, _EPW // _C, step_src, 0)

    def step_dst(j, carry):
      off = pl.multiple_of(base + j * _C, 8)
      pltpu.sync_copy(dst_h.at[pl.ds(off, _C)], idx_v)
      pltpu.async_copy(pos_h.at[idx_v], rows_v, sem).wait()
      pltpu.sync_copy(rows_v, pd_h.at[pl.ds(off, _C)])
      pltpu.sync_copy(ones_v, acc_sh.at[idx_v], add=True)
      return carry
    lax.fori_loop(0, _EPW // _C, step_dst, 0)

    plsc.subcore_barrier()
    out_off = pl.multiple_of(c * _NP + s * _RPT, 8)
    pltpu.sync_copy(acc_sh.at[pl.ds(s * _RPT, _RPT)],
                    deg_h.at[pl.ds(out_off, _RPT)])

  return body


def _sc_pos_deg(pos_pad, src, dst, zero16):
  return _make_sc_pos_deg()(pos_pad, src, dst, zero16)


@functools.lru_cache(maxsize=None)
def _make_sc_gather2():
  kern = functools.partial(
      pl.kernel, mesh=_mesh(),
      compiler_params=pltpu.CompilerParams(use_tc_tiling_on_sc=False),
      out_type=(
          jax.ShapeDtypeStruct((_E, _DP), jnp.float32),
          jax.ShapeDtypeStruct((_E, _DP), jnp.float32),
      ),
      scratch_types=[
          pltpu.VMEM((_C,), jnp.int32),
          pltpu.VMEM((_C, _DP), jnp.float32),
          pltpu.SemaphoreType.DMA,
      ],
  )

  @kern
  def body(tab_h, src_h, dst_h, oa_h, ob_h, idx_v, rows_v, sem):
    c = lax.axis_index("c")
    s = lax.axis_index("s")
    base = (s * _NC + c) * _EPW

    def one(ih, oh):
      def step(j, carry):
        off = pl.multiple_of(base + j * _C, 8)
        pltpu.sync_copy(ih.at[pl.ds(off, _C)], idx_v)
        pltpu.async_copy(tab_h.at[idx_v], rows_v, sem).wait()
        pltpu.sync_copy(rows_v, oh.at[pl.ds(off, _C)])
        return carry
      lax.fori_loop(0, _EPW // _C, step, 0)
    one(src_h, oa_h)
    one(dst_h, ob_h)

  return body


def _sc_gather2(tab, src, dst):
  return _make_sc_gather2()(tab, src, dst)


@functools.lru_cache(maxsize=None)
def _make_sc_scatter():
  kern = functools.partial(
      pl.kernel, mesh=_mesh(),
      compiler_params=pltpu.CompilerParams(use_tc_tiling_on_sc=False),
      out_type=jax.ShapeDtypeStruct((2 * _NP, _DP), jnp.float32),
      scratch_types=[
          pltpu.VMEM((_C,), jnp.int32),
          pltpu.VMEM((_C, _DP), jnp.float32),
          pltpu.VMEM_SHARED((_NP, _DP), jnp.float32),
          pltpu.SemaphoreType.DMA,
      ],
  )

  @kern
  def body(msg_h, dst_h, zero_h, out_h, idx_v, rows_v, acc_sh, sem):
    c = lax.axis_index("c")
    s = lax.axis_index("s")
    base = (s * _NC + c) * _EPW

    pltpu.sync_copy(zero_h, acc_sh.at[pl.ds(s * _RPT, _RPT)])
    plsc.subcore_barrier()

    def step(j, carry):
      off = pl.multiple_of(base + j * _C, 8)
      pltpu.sync_copy(dst_h.at[pl.ds(off, _C)], idx_v)
      pltpu.sync_copy(msg_h.at[pl.ds(off, _C)], rows_v)
      pltpu.sync_copy(rows_v, acc_sh.at[idx_v], add=True)
      return carry
    lax.fori_loop(0, _EPW // _C, step, 0)

    plsc.subcore_barrier()
    out_off = pl.multiple_of(c * _NP + s * _RPT, 8)
    pltpu.sync_copy(acc_sh.at[pl.ds(s * _RPT, _RPT)],
                    out_h.at[pl.ds(out_off, _RPT)])

  return body


def _sc_scatter(msg, dst, zero32):
  return _make_sc_scatter()(msg, dst, zero32)


def _nemb_body(x_ref, w1_ref, b1_ref, w2_ref, b2_ref, h_ref):
    a = jnp.maximum(
        jnp.dot(x_ref[...], w1_ref[...], preferred_element_type=jnp.float32)
        + b1_ref[...], 0.0)
    h_ref[...] = jnp.dot(a, w2_ref[...],
                         preferred_element_type=jnp.float32) + b2_ref[...]


def _efeat_body(ps_ref, pd_ref, ea_ref, w1a_ref, w1b_ref, b1_ref,
                w2_ref, b2_ref, off_ref, es_ref, sh_ref, *, coeff):
    vec = pd_ref[...] - ps_ref[...]
    d2 = jnp.sum(vec * vec, axis=1, keepdims=True)
    d = jnp.sqrt(d2 + 1e-12)
    dist = jnp.exp(coeff * (d - off_ref[...]) ** 2)
    e1 = jnp.maximum(
        jnp.dot(ea_ref[...], w1a_ref[...], preferred_element_type=jnp.float32)
        + jnp.dot(dist, w1b_ref[...], preferred_element_type=jnp.float32)
        + b1_ref[...], 0.0)
    es_ref[...] = jnp.dot(e1, w2_ref[...],
                          preferred_element_type=jnp.float32) + b2_ref[...]
    vhat = vec / d
    vx = vhat[:, 0:1]
    vy = vhat[:, 1:2]
    vz = vhat[:, 2:3]
    c1 = float(np.sqrt(3.0))
    c2 = float(np.sqrt(15.0))
    c3 = float(np.sqrt(5.0) / 2.0)
    vals = [jnp.ones_like(vx), c1 * vx, c1 * vy, c1 * vz,
            c2 * vx * vy, c2 * vy * vz, c3 * (3.0 * vz * vz - 1.0),
            c2 * vx * vz, (c2 / 2.0) * (vx * vx - vy * vy)]
    lane = lax.broadcasted_iota(jnp.int32, sh_ref.shape, 1)
    sh = jnp.zeros(sh_ref.shape, jnp.float32)
    for k, v in enumerate(vals):
        sh = jnp.where(lane == k, v, sh)
    sh_ref[...] = sh


def _dense_body(es_ref, hs_ref, hd_ref, sh_ref, w1_ref, b1_ref,
                w2_ref, b2_ref, out_ref):
    hs = hs_ref[...]
    z = jnp.concatenate([es_ref[...], hs, hd_ref[...]], axis=1)
    hid = jnp.maximum(
        jnp.dot(z, w1_ref[...], preferred_element_type=jnp.float32)
        + b1_ref[...], 0.0)
    t = jnp.dot(hid, w2_ref[...],
                preferred_element_type=jnp.float32) + b2_ref[...]
    sh = sh_ref[...]
    msg = sh[:, 0:1] * t[:, 0:_DP]
    for m in range(1, 9):
        msg = msg + sh[:, m:m + 1] * t[:, m * _DP:(m + 1) * _DP]
    out_ref[...] = msg * hs


def _upd_body(h_ref, p0_ref, p1_ref, g0_ref, g1_ref, ho_ref):
    deg = g0_ref[0, :, 0:1] + g1_ref[0, :, 0:1]
    agg = (p0_ref[0] + p1_ref[0]) / jnp.maximum(deg, 1.0)
    ho_ref[...] = h_ref[...] + agg


def _final_body(h_ref, w_ref, b_ref, o_ref):
    o_ref[...] = jnp.dot(h_ref[...], w_ref[...],
                         preferred_element_type=jnp.float32) + b_ref[...]


def _wspec(shape):
    nd = len(shape)
    return pl.BlockSpec(shape, lambda i: (0,) * nd)


def kernel(x, pos, edge_index, edge_attr, Wn1, bn1, Wn2, bn2, We1, be1,
           We2, be2, Wm1, bm1, Wm2, bm2, Wout, bout):
    f32 = jnp.float32
    src = edge_index[0]
    dst = edge_index[1]

    pos_pad = jnp.zeros((_N, _PW), f32).at[:, :3].set(pos)
    Wn2p = jnp.pad(Wn2, ((0, 0), (0, _DP - _NS)))
    bn2p = jnp.pad(bn2, (0, _DP - _NS))
    We1a = We1[:4]
    We1b = We1[4:]
    We2p = jnp.pad(We2, ((0, 0), (0, _DP - _NS)))
    be2p = jnp.pad(be2, (0, _DP - _NS))
    ridx = jnp.concatenate([jnp.arange(24), jnp.arange(32, 56),
                            jnp.arange(64, 88)])
    W1p = jnp.zeros((4, 96, 72), f32).at[:, ridx].set(Wm1)
    W2p = jnp.pad(Wm2.reshape(4, 72, 9, _NS),
                  ((0, 0), (0, 0), (0, 0), (0, _DP - _NS))).reshape(4, 72, 288)
    b2p = jnp.pad(bm2.reshape(4, 9, _NS),
                  ((0, 0), (0, 0), (0, _DP - _NS))).reshape(4, 288)
    Woutp = jnp.pad(Wout, ((0, _DP - _NS), (0, 0)))

    offs = np.linspace(0.0, 5.0, _DP, dtype=np.float32)
    coeff = float(-0.5 / (float(offs[1]) - float(offs[0])) ** 2)
    offs_op = jnp.asarray(offs)[None, :]

    zero16 = jnp.zeros((_RPT, _PW), f32)
    zero32 = jnp.zeros((_RPT, _DP), f32)

    h = pl.pallas_call(
        _nemb_body,
        grid=(_N // 1000,),
        in_specs=[pl.BlockSpec((1000, 128), lambda i: (i, 0)),
                  _wspec((128, _NS)), _wspec((1, _NS)),
                  _wspec((_NS, _DP)), _wspec((1, _DP))],
        out_specs=pl.BlockSpec((1000, _DP), lambda i: (i, 0)),
        out_shape=jax.ShapeDtypeStruct((_N, _DP), f32),
    )(x, Wn1, bn1[None], Wn2p, bn2p[None])

    ps, pd, degp = _sc_pos_deg(pos_pad, src, dst, zero16)
    degp = degp.reshape(2, _NP, _PW)

    es, sh = pl.pallas_call(
        functools.partial(_efeat_body, coeff=coeff),
        grid=(_E // _BE,),
        in_specs=[pl.BlockSpec((_BE, _PW), lambda i: (i, 0)),
                  pl.BlockSpec((_BE, _PW), lambda i: (i, 0)),
                  pl.BlockSpec((_BE, 4), lambda i: (i, 0)),
                  _wspec((4, _NS)), _wspec((_DP, _NS)), _wspec((1, _NS)),
                  _wspec((_NS, _DP)), _wspec((1, _DP)), _wspec((1, _DP))],
        out_specs=(pl.BlockSpec((_BE, _DP), lambda i: (i, 0)),
                   pl.BlockSpec((_BE, _PW), lambda i: (i, 0))),
        out_shape=(jax.ShapeDtypeStruct((_E, _DP), f32),
                   jax.ShapeDtypeStruct((_E, _PW), f32)),
    )(ps, pd, edge_attr, We1a, We1b, be1[None], We2p, be2p[None], offs_op)

    for i in range(4):
        hs, hd = _sc_gather2(h, src, dst)
        msg = pl.pallas_call(
            _dense_body,
            grid=(_E // _BE,),
            in_specs=[pl.BlockSpec((_BE, _DP), lambda j: (j, 0)),
                      pl.BlockSpec((_BE, _DP), lambda j: (j, 0)),
                      pl.BlockSpec((_BE, _DP), lambda j: (j, 0)),
                      pl.BlockSpec((_BE, _PW), lambda j: (j, 0)),
                      _wspec((96, 72)), _wspec((1, 72)),
                      _wspec((72, 288)), _wspec((1, 288))],
            out_specs=pl.BlockSpec((_BE, _DP), lambda j: (j, 0)),
            out_shape=jax.ShapeDtypeStruct((_E, _DP), f32),
        )(es, hs, hd, sh, W1p[i], bm1[i][None], W2p[i], b2p[i][None])

        parts = _sc_scatter(msg, dst, zero32).reshape(2, _NP, _DP)

        h = pl.pallas_call(
            _upd_body,
            grid=(_N // _BN,),
            in_specs=[pl.BlockSpec((_BN, _DP), lambda j: (j, 0)),
                      pl.BlockSpec((1, _BN, _DP), lambda j: (0, j, 0)),
                      pl.BlockSpec((1, _BN, _DP), lambda j: (1, j, 0)),
                      pl.BlockSpec((1, _BN, _PW), lambda j: (0, j, 0)),
                      pl.BlockSpec((1, _BN, _PW), lambda j: (1, j, 0))],
            out_specs=pl.BlockSpec((_BN, _DP), lambda j: (j, 0)),
            out_shape=jax.ShapeDtypeStruct((_N, _DP), f32),
        )(h, parts, parts, degp, degp)

    return pl.pallas_call(
        _final_body,
        grid=(_N // _BN,),
        in_specs=[pl.BlockSpec((_BN, _DP), lambda j: (j, 0)),
                  _wspec((_DP, _NS)), _wspec((1, _NS))],
        out_specs=pl.BlockSpec((_BN, _NS), lambda j: (j, 0)),
        out_shape=jax.ShapeDtypeStruct((_N, _NS), f32),
    )(h, Woutp, bout[None])


# speedup vs baseline: 2.9882x; 1.4160x over previous
import functools

import jax
import jax.numpy as jnp
import numpy as np
from jax import lax
from jax.experimental import pallas as pl
from jax.experimental.pallas import tpu as pltpu
from jax.experimental.pallas import tpu_sc as plsc

_N = 10000
_E = 160000
_NS = 24
_DP = 32
_PW = 16
_NC = 2
_NT = 16
_NW = _NC * _NT
_EPW = _E // _NW
_C = 1000
_NP = 10240
_RPT = _NP // _NT

_BE = 2000
_BN = 2000


def _mesh():
    return plsc.VectorSubcoreMesh(core_axis_name="c", subcore_axis_name="s")


@functools.lru_cache(maxsize=None)
def _make_sc_pos_deg():
  kern = functools.partial(
      pl.kernel, mesh=_mesh(),
      compiler_params=pltpu.CompilerParams(use_tc_tiling_on_sc=False),
      out_type=(
          jax.ShapeDtypeStruct((_E, _PW), jnp.float32),
          jax.ShapeDtypeStruct((_E, _PW), jnp.float32),
          jax.ShapeDtypeStruct((2 * _NP, _PW), jnp.float32),
      ),
      scratch_types=[
          pltpu.VMEM((_C,), jnp.int32),
          pltpu.VMEM((_C, _PW), jnp.float32),
          pltpu.VMEM((_C, _PW), jnp.float32),
          pltpu.VMEM_SHARED((_NP, _PW), jnp.float32),
          pltpu.SemaphoreType.DMA,
      ],
  )

  @kern
  def body(pos_h, src_h, dst_h, zero_h, ps_h, pd_h, deg_h,
           idx_v, rows_v, ones_v, acc_sh, sem):
    c = lax.axis_index("c")
    s = lax.axis_index("s")
    wid = s * _NC + c
    base = wid * _EPW

    def fill_ones(i, carry):
      ones_v[i] = jnp.ones((_PW,), jnp.float32)
      return carry
    lax.fori_loop(0, _C, fill_ones, 0)
    pltpu.sync_copy(zero_h, acc_sh.at[pl.ds(s * _RPT, _RPT)])
    plsc.subcore_barrier()

    def step_src(j, carry):
      off = pl.multiple_of(base + j * _C, 8)
      pltpu.sync_copy(src_h.at[pl.ds(off, _C)], idx_v)
      pltpu.async_copy(pos_h.at[idx_v], rows_v, sem).wait()
      pltpu.sync_copy(rows_v, ps_h.at[pl.ds(off, _C)])
      return carry
    lax.fori_loop(0, _EPW // _C, step_src, 0)

    def step_dst(j, carry):
      off = pl.multiple_of(base + j * _C, 8)
      pltpu.sync_copy(dst_h.at[pl.ds(off, _C)], idx_v)
      pltpu.async_copy(pos_h.at[idx_v], rows_v, sem).wait()
      pltpu.sync_copy(rows_v, pd_h.at[pl.ds(off, _C)])
      pltpu.sync_copy(ones_v, acc_sh.at[idx_v], add=True)
      return carry
    lax.fori_loop(0, _EPW // _C, step_dst, 0)

    plsc.subcore_barrier()
    out_off = pl.multiple_of(c * _NP + s * _RPT, 8)
    pltpu.sync_copy(acc_sh.at[pl.ds(s * _RPT, _RPT)],
                    deg_h.at[pl.ds(out_off, _RPT)])

  return body


def _sc_pos_deg(pos_pad, src, dst, zero16):
  return _make_sc_pos_deg()(pos_pad, src, dst, zero16)


@functools.lru_cache(maxsize=None)
def _make_sc_gather2():
  kern = functools.partial(
      pl.kernel, mesh=_mesh(),
      compiler_params=pltpu.CompilerParams(use_tc_tiling_on_sc=False),
      out_type=(
          jax.ShapeDtypeStruct((_E, _DP), jnp.float32),
          jax.ShapeDtypeStruct((_E, _DP), jnp.float32),
      ),
      scratch_types=[
          pltpu.VMEM((_C,), jnp.int32),
          pltpu.VMEM((_C, _DP), jnp.float32),
          pltpu.SemaphoreType.DMA,
      ],
  )

  @kern
  def body(tab_h, src_h, dst_h, oa_h, ob_h, idx_v, rows_v, sem):
    c = lax.axis_index("c")
    s = lax.axis_index("s")
    base = (s * _NC + c) * _EPW

    def one(ih, oh):
      def step(j, carry):
        off = pl.multiple_of(base + j * _C, 8)
        pltpu.sync_copy(ih.at[pl.ds(off, _C)], idx_v)
        pltpu.async_copy(tab_h.at[idx_v], rows_v, sem).wait()
        pltpu.sync_copy(rows_v, oh.at[pl.ds(off, _C)])
        return carry
      lax.fori_loop(0, _EPW // _C, step, 0)
    one(src_h, oa_h)
    one(dst_h, ob_h)

  return body


def _sc_gather2(tab, src, dst):
  return _make_sc_gather2()(tab, src, dst)


@functools.lru_cache(maxsize=None)
def _make_sc_scatter():
  kern = functools.partial(
      pl.kernel, mesh=_mesh(),
      compiler_params=pltpu.CompilerParams(use_tc_tiling_on_sc=False),
      out_type=jax.ShapeDtypeStruct((2 * _NP, _DP), jnp.float32),
      scratch_types=[
          pltpu.VMEM((_C,), jnp.int32),
          pltpu.VMEM((_C, _DP), jnp.float32),
          pltpu.VMEM_SHARED((_NP, _DP), jnp.float32),
          pltpu.SemaphoreType.DMA,
      ],
  )

  @kern
  def body(msg_h, dst_h, zero_h, out_h, idx_v, rows_v, acc_sh, sem):
    c = lax.axis_index("c")
    s = lax.axis_index("s")
    base = (s * _NC + c) * _EPW

    pltpu.sync_copy(zero_h, acc_sh.at[pl.ds(s * _RPT, _RPT)])
    plsc.subcore_barrier()

    def step(j, carry):
      off = pl.multiple_of(base + j * _C, 8)
      pltpu.sync_copy(dst_h.at[pl.ds(off, _C)], idx_v)
      pltpu.sync_copy(msg_h.at[pl.ds(off, _C)], rows_v)
      pltpu.sync_copy(rows_v, acc_sh.at[idx_v], add=True)
      return carry
    lax.fori_loop(0, _EPW // _C, step, 0)

    plsc.subcore_barrier()
    out_off = pl.multiple_of(c * _NP + s * _RPT, 8)
    pltpu.sync_copy(acc_sh.at[pl.ds(s * _RPT, _RPT)],
                    out_h.at[pl.ds(out_off, _RPT)])

  return body


def _sc_scatter(msg, dst, zero32):
  return _make_sc_scatter()(msg, dst, zero32)


def _nemb_body(x_ref, w1_ref, b1_ref, w2_ref, b2_ref, h_ref):
    a = jnp.maximum(
        jnp.dot(x_ref[...], w1_ref[...], preferred_element_type=jnp.float32)
        + b1_ref[...], 0.0)
    h_ref[...] = jnp.dot(a, w2_ref[...],
                         preferred_element_type=jnp.float32) + b2_ref[...]


def _efeat_body(ps_ref, pd_ref, ea_ref, w1a_ref, w1b_ref, b1_ref,
                w2_ref, b2_ref, off_ref, aa_ref, ab_ref, ac_ref, c0_ref,
                es_ref, sh_ref, *, coeff):
    vec = pd_ref[...] - ps_ref[...]
    d2 = jnp.sum(vec * vec, axis=1, keepdims=True)
    d = jnp.sqrt(d2 + 1e-12)
    dist = jnp.exp(coeff * (d - off_ref[...]) ** 2)
    e1 = jnp.maximum(
        jnp.dot(ea_ref[...], w1a_ref[...], preferred_element_type=jnp.float32)
        + jnp.dot(dist, w1b_ref[...], preferred_element_type=jnp.float32)
        + b1_ref[...], 0.0)
    es_ref[...] = jnp.dot(e1, w2_ref[...],
                          preferred_element_type=jnp.float32) + b2_ref[...]
    vhat = vec / d
    ga = jnp.dot(vhat, aa_ref[...], preferred_element_type=jnp.float32)
    gb = jnp.dot(vhat, ab_ref[...], preferred_element_type=jnp.float32)
    gc = jnp.dot(vhat, ac_ref[...], preferred_element_type=jnp.float32)
    sh_ref[...] = ga * gb + gc + c0_ref[...]


def _dense_body(es_ref, hs_ref, hd_ref, sh_ref, w1a_ref, w1b_ref, w1c_ref,
                b1_ref, w2_ref, b2_ref, bm_ref, fm_ref, out_ref):
    hs = hs_ref[...]
    hid = jnp.maximum(
        jnp.dot(es_ref[...], w1a_ref[...], preferred_element_type=jnp.float32)
        + jnp.dot(hs, w1b_ref[...], preferred_element_type=jnp.float32)
        + jnp.dot(hd_ref[...], w1c_ref[...], preferred_element_type=jnp.float32)
        + b1_ref[...], 0.0)
    t = jnp.dot(hid, w2_ref[...],
                preferred_element_type=jnp.float32) + b2_ref[...]
    s = jnp.dot(sh_ref[...], bm_ref[...],
                preferred_element_type=jnp.float32)
    msg = jnp.dot(s * t, fm_ref[...],
                  preferred_element_type=jnp.float32)
    out_ref[...] = msg * hs


def _upd_body(h_ref, p0_ref, p1_ref, g0_ref, g1_ref, ho_ref):
    deg = g0_ref[0, :, 0:1] + g1_ref[0, :, 0:1]
    agg = (p0_ref[0] + p1_ref[0]) / jnp.maximum(deg, 1.0)
    ho_ref[...] = h_ref[...] + agg


def _final_body(h_ref, w_ref, b_ref, o_ref):
    o_ref[...] = jnp.dot(h_ref[...], w_ref[...],
                         preferred_element_type=jnp.float32) + b_ref[...]


def _wspec(shape):
    nd = len(shape)
    return pl.BlockSpec(shape, lambda i: (0,) * nd)


def kernel(x, pos, edge_index, edge_attr, Wn1, bn1, Wn2, bn2, We1, be1,
           We2, be2, Wm1, bm1, Wm2, bm2, Wout, bout):
    f32 = jnp.float32
    src = edge_index[0]
    dst = edge_index[1]

    pos_pad = jnp.zeros((_N, _PW), f32).at[:, :3].set(pos)
    Wn2p = jnp.pad(Wn2, ((0, 0), (0, _DP - _NS)))
    bn2p = jnp.pad(bn2, (0, _DP - _NS))
    We1a = We1[:4]
    We1b = We1[4:]
    We2p = jnp.pad(We2, ((0, 0), (0, _DP - _NS)))
    be2p = jnp.pad(be2, (0, _DP - _NS))
    W1a = jnp.pad(Wm1[:, 0:24], ((0, 0), (0, 8), (0, 0)))
    W1b = jnp.pad(Wm1[:, 24:48], ((0, 0), (0, 8), (0, 0)))
    W1c = jnp.pad(Wm1[:, 48:72], ((0, 0), (0, 8), (0, 0)))
    lane = np.arange(288)
    Bm = jnp.asarray((lane[None, :] // _DP == np.arange(16)[:, None])
                     .astype(np.float32))
    Fm = jnp.asarray((lane[:, None] % _DP == np.arange(_DP)[None, :])
                     .astype(np.float32))
    W2p = jnp.pad(Wm2.reshape(4, 72, 9, _NS),
                  ((0, 0), (0, 0), (0, 0), (0, _DP - _NS))).reshape(4, 72, 288)
    b2p = jnp.pad(bm2.reshape(4, 9, _NS),
                  ((0, 0), (0, 0), (0, _DP - _NS))).reshape(4, 288)
    Woutp = jnp.pad(Wout, ((0, _DP - _NS), (0, 0)))

    offs = np.linspace(0.0, 5.0, _DP, dtype=np.float32)
    coeff = float(-0.5 / (float(offs[1]) - float(offs[0])) ** 2)
    offs_op = jnp.asarray(offs)[None, :]

    c1 = float(np.sqrt(3.0))
    c2 = float(np.sqrt(15.0))
    c3 = float(np.sqrt(5.0) / 2.0)
    Aa = np.zeros((16, 16), np.float32)
    Ab = np.zeros((16, 16), np.float32)
    Ac = np.zeros((16, 16), np.float32)
    c0 = np.zeros((1, 16), np.float32)
    X, Y, Z = 0, 1, 2
    c0[0, 0] = 1.0
    Ac[X, 1] = c1
    Ac[Y, 2] = c1
    Ac[Z, 3] = c1
    Aa[X, 4] = 1.0
    Ab[Y, 4] = c2
    Aa[Y, 5] = 1.0
    Ab[Z, 5] = c2
    Aa[Z, 6] = 1.0
    Ab[Z, 6] = 3.0 * c3
    c0[0, 6] = -c3
    Aa[X, 7] = 1.0
    Ab[Z, 7] = c2
    Aa[X, 8] = 1.0
    Aa[Y, 8] = 1.0
    Ab[X, 8] = c2 / 2.0
    Ab[Y, 8] = -c2 / 2.0
    Aa, Ab, Ac, c0 = map(jnp.asarray, (Aa, Ab, Ac, c0))

    zero16 = jnp.zeros((_RPT, _PW), f32)
    zero32 = jnp.zeros((_RPT, _DP), f32)

    h = pl.pallas_call(
        _nemb_body,
        grid=(_N // 1000,),
        in_specs=[pl.BlockSpec((1000, 128), lambda i: (i, 0)),
                  _wspec((128, _NS)), _wspec((1, _NS)),
                  _wspec((_NS, _DP)), _wspec((1, _DP))],
        out_specs=pl.BlockSpec((1000, _DP), lambda i: (i, 0)),
        out_shape=jax.ShapeDtypeStruct((_N, _DP), f32),
    )(x, Wn1, bn1[None], Wn2p, bn2p[None])

    ps, pd, degp = _sc_pos_deg(pos_pad, src, dst, zero16)
    degp = degp.reshape(2, _NP, _PW)

    es, sh = pl.pallas_call(
        functools.partial(_efeat_body, coeff=coeff),
        grid=(_E // _BE,),
        in_specs=[pl.BlockSpec((_BE, _PW), lambda i: (i, 0)),
                  pl.BlockSpec((_BE, _PW), lambda i: (i, 0)),
                  pl.BlockSpec((_BE, 4), lambda i: (i, 0)),
                  _wspec((4, _NS)), _wspec((_DP, _NS)), _wspec((1, _NS)),
                  _wspec((_NS, _DP)), _wspec((1, _DP)), _wspec((1, _DP)),
                  _wspec((16, 16)), _wspec((16, 16)), _wspec((16, 16)),
                  _wspec((1, 16))],
        out_specs=(pl.BlockSpec((_BE, _DP), lambda i: (i, 0)),
                   pl.BlockSpec((_BE, _PW), lambda i: (i, 0))),
        out_shape=(jax.ShapeDtypeStruct((_E, _DP), f32),
                   jax.ShapeDtypeStruct((_E, _PW), f32)),
    )(ps, pd, edge_attr, We1a, We1b, be1[None], We2p, be2p[None], offs_op,
      Aa, Ab, Ac, c0)

    for i in range(4):
        hs, hd = _sc_gather2(h, src, dst)
        msg = pl.pallas_call(
            _dense_body,
            grid=(_E // _BE,),
            in_specs=[pl.BlockSpec((_BE, _DP), lambda j: (j, 0)),
                      pl.BlockSpec((_BE, _DP), lambda j: (j, 0)),
                      pl.BlockSpec((_BE, _DP), lambda j: (j, 0)),
                      pl.BlockSpec((_BE, _PW), lambda j: (j, 0)),
                      _wspec((_DP, 72)), _wspec((_DP, 72)), _wspec((_DP, 72)),
                      _wspec((1, 72)), _wspec((72, 288)), _wspec((1, 288)),
                      _wspec((16, 288)), _wspec((288, _DP))],
            out_specs=pl.BlockSpec((_BE, _DP), lambda j: (j, 0)),
            out_shape=jax.ShapeDtypeStruct((_E, _DP), f32),
        )(es, hs, hd, sh, W1a[i], W1b[i], W1c[i], bm1[i][None],
          W2p[i], b2p[i][None], Bm, Fm)

        parts = _sc_scatter(msg, dst, zero32).reshape(2, _NP, _DP)

        h = pl.pallas_call(
            _upd_body,
            grid=(_N // _BN,),
            in_specs=[pl.BlockSpec((_BN, _DP), lambda j: (j, 0)),
                      pl.BlockSpec((1, _BN, _DP), lambda j: (0, j, 0)),
                      pl.BlockSpec((1, _BN, _DP), lambda j: (1, j, 0)),
                      pl.BlockSpec((1, _BN, _PW), lambda j: (0, j, 0)),
                      pl.BlockSpec((1, _BN, _PW), lambda j: (1, j, 0))],
            out_specs=pl.BlockSpec((_BN, _DP), lambda j: (j, 0)),
            out_shape=jax.ShapeDtypeStruct((_N, _DP), f32),
        )(h, parts, parts, degp, degp)

    return pl.pallas_call(
        _final_body,
        grid=(_N // _BN,),
        in_specs=[pl.BlockSpec((_BN, _DP), lambda j: (j, 0)),
                  _wspec((_DP, _NS)), _wspec((1, _NS))],
        out_specs=pl.BlockSpec((_BN, _NS), lambda j: (j, 0)),
        out_shape=jax.ShapeDtypeStruct((_N, _NS), f32),
    )(h, Woutp, bout[None])


# speedup vs baseline: 3.6629x; 1.2258x over previous
import functools

import jax
import jax.numpy as jnp
import numpy as np
from jax import lax
from jax.experimental import pallas as pl
from jax.experimental.pallas import tpu as pltpu
from jax.experimental.pallas import tpu_sc as plsc

_N = 10000
_E = 160000
_E2 = 163840
_NS = 24
_DP = 32
_PW = 16
_NC = 2
_NT = 16
_NW = _NC * _NT
_EPW = _E2 // _NW
_C = 1024
_NP = 10240
_RPT = _NP // _NT

_BE = 2048
_BN = 2000


def _mesh():
    return plsc.VectorSubcoreMesh(core_axis_name="c", subcore_axis_name="s")


@functools.lru_cache(maxsize=None)
def _make_sc_pos_deg():
  kern = functools.partial(
      pl.kernel, mesh=_mesh(),
      compiler_params=pltpu.CompilerParams(use_tc_tiling_on_sc=False),
      out_type=(
          jax.ShapeDtypeStruct((_E2, _DP), jnp.float32),
          jax.ShapeDtypeStruct((_E2, _DP), jnp.float32),
          jax.ShapeDtypeStruct((2 * _NP, _PW), jnp.float32),
      ),
      scratch_types=[
          pltpu.VMEM((_C,), jnp.int32),
          pltpu.VMEM((_C, _DP), jnp.float32),
          pltpu.VMEM((_C, _PW), jnp.float32),
          pltpu.VMEM_SHARED((_NP, _PW), jnp.float32),
          pltpu.SemaphoreType.DMA,
      ],
  )

  @kern
  def body(pos_h, src_h, dst_h, zero_h, ps_h, pd_h, deg_h,
           idx_v, rows_v, ones_v, acc_sh, sem):
    c = lax.axis_index("c")
    s = lax.axis_index("s")
    base = (s * _NC + c) * _EPW

    def fill_ones(i, carry):
      ones_v[i] = jnp.ones((_PW,), jnp.float32)
      return carry
    lax.fori_loop(0, _C, fill_ones, 0)
    pltpu.sync_copy(zero_h, acc_sh.at[pl.ds(s * _RPT, _RPT)])
    plsc.subcore_barrier()

    def step_src(j, carry):
      off = pl.multiple_of(base + j * _C, 8)
      pltpu.sync_copy(src_h.at[pl.ds(off, _C)], idx_v)
      pltpu.async_copy(pos_h.at[idx_v], rows_v, sem).wait()
      pltpu.sync_copy(rows_v, ps_h.at[pl.ds(off, _C)])
      return carry
    lax.fori_loop(0, _EPW // _C, step_src, 0)

    def step_dst(j, carry):
      off = pl.multiple_of(base + j * _C, 8)
      pltpu.sync_copy(dst_h.at[pl.ds(off, _C)], idx_v)
      pltpu.async_copy(pos_h.at[idx_v], rows_v, sem).wait()
      pltpu.sync_copy(rows_v, pd_h.at[pl.ds(off, _C)])
      pltpu.sync_copy(ones_v, acc_sh.at[idx_v], add=True)
      return carry
    lax.fori_loop(0, _EPW // _C, step_dst, 0)

    plsc.subcore_barrier()
    out_off = pl.multiple_of(c * _NP + s * _RPT, 8)
    pltpu.sync_copy(acc_sh.at[pl.ds(s * _RPT, _RPT)],
                    deg_h.at[pl.ds(out_off, _RPT)])

  return body


def _sc_pos_deg(pos_pad, src, dst, zero16):
  return _make_sc_pos_deg()(pos_pad, src, dst, zero16)


@functools.lru_cache(maxsize=None)
def _make_sc_gather2():
  kern = functools.partial(
      pl.kernel, mesh=_mesh(),
      compiler_params=pltpu.CompilerParams(use_tc_tiling_on_sc=False),
      out_type=(
          jax.ShapeDtypeStruct((_E2, _DP), jnp.float32),
          jax.ShapeDtypeStruct((_E2, _DP), jnp.float32),
      ),
      scratch_types=[
          pltpu.VMEM((_C,), jnp.int32),
          pltpu.VMEM((_C, _DP), jnp.float32),
          pltpu.SemaphoreType.DMA,
      ],
  )

  @kern
  def body(tab_h, src_h, dst_h, oa_h, ob_h, idx_v, rows_v, sem):
    c = lax.axis_index("c")
    s = lax.axis_index("s")
    base = (s * _NC + c) * _EPW

    def one(ih, oh):
      def step(j, carry):
        off = pl.multiple_of(base + j * _C, 8)
        pltpu.sync_copy(ih.at[pl.ds(off, _C)], idx_v)
        pltpu.async_copy(tab_h.at[idx_v], rows_v, sem).wait()
        pltpu.sync_copy(rows_v, oh.at[pl.ds(off, _C)])
        return carry
      lax.fori_loop(0, _EPW // _C, step, 0)
    one(src_h, oa_h)
    one(dst_h, ob_h)

  return body


def _sc_gather2(tab, src, dst):
  return _make_sc_gather2()(tab, src, dst)


@functools.lru_cache(maxsize=None)
def _make_sc_scatter():
  kern = functools.partial(
      pl.kernel, mesh=_mesh(),
      compiler_params=pltpu.CompilerParams(use_tc_tiling_on_sc=False),
      out_type=jax.ShapeDtypeStruct((2 * _NP, _DP), jnp.float32),
      scratch_types=[
          pltpu.VMEM((_C,), jnp.int32),
          pltpu.VMEM((_C, _DP), jnp.float32),
          pltpu.VMEM_SHARED((_NP, _DP), jnp.float32),
          pltpu.SemaphoreType.DMA,
      ],
  )

  @kern
  def body(msg_h, dst_h, zero_h, out_h, idx_v, rows_v, acc_sh, sem):
    c = lax.axis_index("c")
    s = lax.axis_index("s")
    base = (s * _NC + c) * _EPW

    pltpu.sync_copy(zero_h, acc_sh.at[pl.ds(s * _RPT, _RPT)])
    plsc.subcore_barrier()

    def step(j, carry):
      off = pl.multiple_of(base + j * _C, 8)
      pltpu.sync_copy(dst_h.at[pl.ds(off, _C)], idx_v)
      pltpu.sync_copy(msg_h.at[pl.ds(off, _C)], rows_v)
      pltpu.sync_copy(rows_v, acc_sh.at[idx_v], add=True)
      return carry
    lax.fori_loop(0, _EPW // _C, step, 0)

    plsc.subcore_barrier()
    out_off = pl.multiple_of(c * _NP + s * _RPT, 8)
    pltpu.sync_copy(acc_sh.at[pl.ds(s * _RPT, _RPT)],
                    out_h.at[pl.ds(out_off, _RPT)])

  return body


def _sc_scatter(msg, dst, zero32):
  return _make_sc_scatter()(msg, dst, zero32)


def _nemb_body(x_ref, w1_ref, b1_ref, w2_ref, b2_ref, h_ref):
    a = jnp.maximum(
        jnp.dot(x_ref[...], w1_ref[...], preferred_element_type=jnp.float32)
        + b1_ref[...], 0.0)
    h_ref[...] = jnp.dot(a, w2_ref[...],
                         preferred_element_type=jnp.float32) + b2_ref[...]


def _efeat_body(ps_ref, pd_ref, ea_ref, w1a_ref, w1b_ref, b1_ref,
                w2_ref, b2_ref, off_ref, aa_ref, ab_ref, ac_ref, c0_ref,
                g4_ref, e4_ref, es_ref, sh_ref, *, coeff):
    f32 = jnp.float32
    vec4 = pd_ref[...] - ps_ref[...]
    d2 = jnp.dot(vec4 * vec4, g4_ref[...],
                 preferred_element_type=f32)
    d = jnp.sqrt(d2 + 1e-12)
    d4e = jnp.dot(d, e4_ref[...], preferred_element_type=f32)
    r4e = jnp.dot(1.0 / d, e4_ref[...], preferred_element_type=f32)
    dist4 = jnp.exp(coeff * (d4e - off_ref[...]) ** 2)
    e1 = jnp.maximum(
        jnp.dot(ea_ref[...], w1a_ref[...], preferred_element_type=f32)
        + jnp.dot(dist4, w1b_ref[...], preferred_element_type=f32)
        + b1_ref[...], 0.0)
    es_ref[...] = jnp.dot(e1, w2_ref[...],
                          preferred_element_type=f32) + b2_ref[...]
    vhat4 = vec4 * r4e
    ga = jnp.dot(vhat4, aa_ref[...], preferred_element_type=f32)
    gb = jnp.dot(vhat4, ab_ref[...], preferred_element_type=f32)
    gc = jnp.dot(vhat4, ac_ref[...], preferred_element_type=f32)
    sh_ref[...] = ga * gb + gc + c0_ref[...]


def _dense_body(es_ref, hs_ref, hd_ref, sh_ref, w1_ref, b1_ref,
                w2_ref, b2_ref, bm_ref, fm_ref, out_ref):
    hs4 = hs_ref[...]
    z4 = jnp.concatenate([es_ref[...], hs4, hd_ref[...]], axis=1)
    hid4 = jnp.maximum(
        jnp.dot(z4, w1_ref[...], preferred_element_type=jnp.float32)
        + b1_ref[...], 0.0)
    t4 = jnp.dot(hid4, w2_ref[...],
                 preferred_element_type=jnp.float32) + b2_ref[...]
    s4 = jnp.dot(sh_ref[...], bm_ref[...],
                 preferred_element_type=jnp.float32)
    msg4 = jnp.dot(s4 * t4, fm_ref[...],
                   preferred_element_type=jnp.float32)
    out_ref[...] = msg4 * hs4


def _upd_body(h_ref, p0_ref, p1_ref, g0_ref, g1_ref, ho_ref):
    deg = g0_ref[0, :, 0:1] + g1_ref[0, :, 0:1]
    agg = (p0_ref[0] + p1_ref[0]) / jnp.maximum(deg, 1.0)
    ho_ref[...] = h_ref[...] + agg


def _final_body(h_ref, w_ref, b_ref, o_ref):
    o_ref[...] = jnp.dot(h_ref[...], w_ref[...],
                         preferred_element_type=jnp.float32) + b_ref[...]


def _wspec(shape):
    nd = len(shape)
    return pl.BlockSpec(shape, lambda i: (0,) * nd)


def kernel(x, pos, edge_index, edge_attr, Wn1, bn1, Wn2, bn2, We1, be1,
           We2, be2, Wm1, bm1, Wm2, bm2, Wout, bout):
    f32 = jnp.float32
    src = jnp.pad(edge_index[0], (0, _E2 - _E), constant_values=_N)
    dst = jnp.pad(edge_index[1], (0, _E2 - _E), constant_values=_N)
    ea4 = jnp.pad(edge_attr, ((0, _E2 - _E), (0, 0))).reshape(_E2 // 4, 16)

    eye4 = np.eye(4, dtype=np.float32)
    pos_pad = jnp.zeros((_NP, _DP), f32).at[:_N, :3].set(pos)
    Wn2p = jnp.pad(Wn2, ((0, 0), (0, _DP - _NS)))
    bn2p = jnp.pad(bn2, (0, _DP - _NS))
    We1a4 = jnp.kron(eye4, jnp.pad(We1[:4], ((0, 0), (0, 8))))
    We1b4 = jnp.kron(eye4, jnp.pad(We1[4:], ((0, 0), (0, 8))))
    be1_4 = jnp.tile(jnp.pad(be1, (0, 8)), 4)
    We2p4 = jnp.kron(eye4, jnp.pad(We2, ((0, 8), (0, 8))))
    be2p_4 = jnp.tile(jnp.pad(be2, (0, 8)), 4)
    W1blk = jnp.zeros((4, 384, 288), f32)
    for slot in range(4):
        r = slot * 32
        cidx = slot * 72
        W1blk = W1blk.at[:, r:r + 24, cidx:cidx + 72].set(Wm1[:, 0:24])
        W1blk = W1blk.at[:, 128 + r:128 + r + 24, cidx:cidx + 72].set(
            Wm1[:, 24:48])
        W1blk = W1blk.at[:, 256 + r:256 + r + 24, cidx:cidx + 72].set(
            Wm1[:, 48:72])
    b1_4 = jnp.tile(bm1, (1, 4))
    W2p = jnp.pad(Wm2.reshape(4, 72, 9, _NS),
                  ((0, 0), (0, 0), (0, 0), (0, _DP - _NS))).reshape(4, 72, 288)
    b2p = jnp.pad(bm2.reshape(4, 9, _NS),
                  ((0, 0), (0, 0), (0, _DP - _NS))).reshape(4, 288)
    W2p4 = jnp.stack([jnp.kron(eye4, W2p[i]) for i in range(4)])
    b2_4 = jnp.tile(b2p, (1, 4))
    Woutp = jnp.pad(Wout, ((0, _DP - _NS), (0, 0)))
    lanes = np.arange(288)
    Bm = (lanes[None, :] // _DP == np.arange(16)[:, None]).astype(np.float32)
    Fm = (lanes[:, None] % _DP == np.arange(_DP)[None, :]).astype(np.float32)
    Bm4 = jnp.asarray(np.kron(eye4, np.pad(Bm, ((0, 16), (0, 0)))))
    F4 = jnp.asarray(np.kron(eye4, Fm))
    G4 = jnp.asarray(np.kron(eye4, np.ones((32, 1), np.float32)))
    E4 = jnp.asarray(np.kron(eye4, np.ones((1, 32), np.float32)))

    offs = np.linspace(0.0, 5.0, _DP, dtype=np.float32)
    coeff = float(-0.5 / (float(offs[1]) - float(offs[0])) ** 2)
    off4 = jnp.asarray(np.tile(offs, 4))[None, :]

    c1 = float(np.sqrt(3.0))
    c2 = float(np.sqrt(15.0))
    c3 = float(np.sqrt(5.0) / 2.0)
    Aa = np.zeros((16, 16), np.float32)
    Ab = np.zeros((16, 16), np.float32)
    Ac = np.zeros((16, 16), np.float32)
    c0 = np.zeros((1, 16), np.float32)
    X, Y, Z = 0, 1, 2
    c0[0, 0] = 1.0
    Ac[X, 1] = c1
    Ac[Y, 2] = c1
    Ac[Z, 3] = c1
    Aa[X, 4] = 1.0
    Ab[Y, 4] = c2
    Aa[Y, 5] = 1.0
    Ab[Z, 5] = c2
    Aa[Z, 6] = 1.0
    Ab[Z, 6] = 3.0 * c3
    c0[0, 6] = -c3
    Aa[X, 7] = 1.0
    Ab[Z, 7] = c2
    Aa[X, 8] = 1.0
    Aa[Y, 8] = 1.0
    Ab[X, 8] = c2 / 2.0
    Ab[Y, 8] = -c2 / 2.0
    def _blk32(m):
        return jnp.asarray(np.kron(eye4, np.pad(m, ((0, 16), (0, 16)))))
    Aa4, Ab4, Ac4 = _blk32(Aa), _blk32(Ab), _blk32(Ac)
    c0_4 = jnp.asarray(np.tile(np.pad(c0, ((0, 0), (0, 16))), (1, 4)))

    zero16 = jnp.zeros((_RPT, _PW), f32)
    zero32 = jnp.zeros((_RPT, _DP), f32)

    h = pl.pallas_call(
        _nemb_body,
        grid=(_N // 1000,),
        in_specs=[pl.BlockSpec((1000, 128), lambda i: (i, 0)),
                  _wspec((128, _NS)), _wspec((1, _NS)),
                  _wspec((_NS, _DP)), _wspec((1, _DP))],
        out_specs=pl.BlockSpec((1000, _DP), lambda i: (i, 0)),
        out_shape=jax.ShapeDtypeStruct((_N, _DP), f32),
    )(x, Wn1, bn1[None], Wn2p, bn2p[None])

    ps, pd, degp = _sc_pos_deg(pos_pad, src, dst, zero16)
    degp = degp.reshape(2, _NP, _PW)
    ps4 = ps.reshape(_E2 // 4, 128)
    pd4 = pd.reshape(_E2 // 4, 128)

    es4, sh4 = pl.pallas_call(
        functools.partial(_efeat_body, coeff=coeff),
        grid=(_E2 // _BE,),
        in_specs=[pl.BlockSpec((_BE // 4, 128), lambda i: (i, 0)),
                  pl.BlockSpec((_BE // 4, 128), lambda i: (i, 0)),
                  pl.BlockSpec((_BE // 4, 16), lambda i: (i, 0)),
                  _wspec((16, 128)), _wspec((128, 128)), _wspec((1, 128)),
                  _wspec((128, 128)), _wspec((1, 128)), _wspec((1, 128)),
                  _wspec((128, 128)), _wspec((128, 128)), _wspec((128, 128)),
                  _wspec((1, 128)), _wspec((128, 4)), _wspec((4, 128))],
        out_specs=(pl.BlockSpec((_BE // 4, 128), lambda i: (i, 0)),
                   pl.BlockSpec((_BE // 4, 128), lambda i: (i, 0))),
        out_shape=(jax.ShapeDtypeStruct((_E2 // 4, 128), f32),
                   jax.ShapeDtypeStruct((_E2 // 4, 128), f32)),
    )(ps4, pd4, ea4, We1a4, We1b4, be1_4[None], We2p4, be2p_4[None], off4,
      Aa4, Ab4, Ac4, c0_4, G4, E4)

    hp = jnp.zeros((_NP, _DP), f32).at[:_N].set(h)
    for i in range(4):
        hs, hd = _sc_gather2(hp, src, dst)
        hs4 = hs.reshape(_E2 // 4, 128)
        hd4 = hd.reshape(_E2 // 4, 128)
        msg4 = pl.pallas_call(
            _dense_body,
            grid=(_E2 // _BE,),
            in_specs=[pl.BlockSpec((_BE // 4, 128), lambda j: (j, 0)),
                      pl.BlockSpec((_BE // 4, 128), lambda j: (j, 0)),
                      pl.BlockSpec((_BE // 4, 128), lambda j: (j, 0)),
                      pl.BlockSpec((_BE // 4, 128), lambda j: (j, 0)),
                      _wspec((384, 288)), _wspec((1, 288)),
                      _wspec((288, 1152)), _wspec((1, 1152)),
                      _wspec((128, 1152)), _wspec((1152, 128))],
            out_specs=pl.BlockSpec((_BE // 4, 128), lambda j: (j, 0)),
            out_shape=jax.ShapeDtypeStruct((_E2 // 4, 128), f32),
        )(es4, hs4, hd4, sh4, W1blk[i], b1_4[i][None],
          W2p4[i], b2_4[i][None], Bm4, F4)
        msg = msg4.reshape(_E2, _DP)

        parts = _sc_scatter(msg, dst, zero32).reshape(2, _NP, _DP)

        h = pl.pallas_call(
            _upd_body,
            grid=(_N // _BN,),
            in_specs=[pl.BlockSpec((_BN, _DP), lambda j: (j, 0)),
                      pl.BlockSpec((1, _BN, _DP), lambda j: (0, j, 0)),
                      pl.BlockSpec((1, _BN, _DP), lambda j: (1, j, 0)),
                      pl.BlockSpec((1, _BN, _PW), lambda j: (0, j, 0)),
                      pl.BlockSpec((1, _BN, _PW), lambda j: (1, j, 0))],
            out_specs=pl.BlockSpec((_BN, _DP), lambda j: (j, 0)),
            out_shape=jax.ShapeDtypeStruct((_N, _DP), f32),
        )(h, parts, parts, degp, degp)
        if i < 3:
            hp = jnp.zeros((_NP, _DP), f32).at[:_N].set(h)

    return pl.pallas_call(
        _final_body,
        grid=(_N // _BN,),
        in_specs=[pl.BlockSpec((_BN, _DP), lambda j: (j, 0)),
                  _wspec((_DP, _NS)), _wspec((1, _NS))],
        out_specs=pl.BlockSpec((_BN, _NS), lambda j: (j, 0)),
        out_shape=jax.ShapeDtypeStruct((_N, _NS), f32),
    )(h, Woutp, bout[None])


# speedup vs baseline: 3.8413x; 1.0487x over previous
import functools

import jax
import jax.numpy as jnp
import numpy as np
from jax import lax
from jax.experimental import pallas as pl
from jax.experimental.pallas import tpu as pltpu
from jax.experimental.pallas import tpu_sc as plsc

_N = 10000
_E = 160000
_E2 = 163840
_NS = 24
_DP = 32
_PW = 16
_NC = 2
_NT = 16
_NW = _NC * _NT
_EPW = _E2 // _NW
_C = 1024
_NP = 10240
_RPT = _NP // _NT

_BE = 2048
_BN = 2000


def _mesh():
    return plsc.VectorSubcoreMesh(core_axis_name="c", subcore_axis_name="s")


@functools.lru_cache(maxsize=None)
def _make_sc_pos_deg():
  kern = functools.partial(
      pl.kernel, mesh=_mesh(),
      compiler_params=pltpu.CompilerParams(use_tc_tiling_on_sc=False),
      out_type=(
          jax.ShapeDtypeStruct((_E2, _DP), jnp.float32),
          jax.ShapeDtypeStruct((_E2, _DP), jnp.float32),
          jax.ShapeDtypeStruct((2 * _NP, _PW), jnp.float32),
      ),
      scratch_types=[
          pltpu.VMEM((_C,), jnp.int32),
          pltpu.VMEM((_C, _DP), jnp.float32),
          pltpu.VMEM((_C, _PW), jnp.float32),
          pltpu.VMEM_SHARED((_NP, _PW), jnp.float32),
          pltpu.SemaphoreType.DMA,
      ],
  )

  @kern
  def body(pos_h, src_h, dst_h, zero_h, ps_h, pd_h, deg_h,
           idx_v, rows_v, ones_v, acc_sh, sem):
    c = lax.axis_index("c")
    s = lax.axis_index("s")
    base = (s * _NC + c) * _EPW

    def fill_ones(i, carry):
      ones_v[i] = jnp.ones((_PW,), jnp.float32)
      return carry
    lax.fori_loop(0, _C, fill_ones, 0)
    pltpu.sync_copy(zero_h, acc_sh.at[pl.ds(s * _RPT, _RPT)])
    plsc.subcore_barrier()

    def step_src(j, carry):
      off = pl.multiple_of(base + j * _C, 8)
      pltpu.sync_copy(src_h.at[pl.ds(off, _C)], idx_v)
      pltpu.async_copy(pos_h.at[idx_v], rows_v, sem).wait()
      pltpu.sync_copy(rows_v, ps_h.at[pl.ds(off, _C)])
      return carry
    lax.fori_loop(0, _EPW // _C, step_src, 0)

    def step_dst(j, carry):
      off = pl.multiple_of(base + j * _C, 8)
      pltpu.sync_copy(dst_h.at[pl.ds(off, _C)], idx_v)
      pltpu.async_copy(pos_h.at[idx_v], rows_v, sem).wait()
      pltpu.sync_copy(rows_v, pd_h.at[pl.ds(off, _C)])
      pltpu.sync_copy(ones_v, acc_sh.at[idx_v], add=True)
      return carry
    lax.fori_loop(0, _EPW // _C, step_dst, 0)

    plsc.subcore_barrier()
    out_off = pl.multiple_of(c * _NP + s * _RPT, 8)
    pltpu.sync_copy(acc_sh.at[pl.ds(s * _RPT, _RPT)],
                    deg_h.at[pl.ds(out_off, _RPT)])

  return body


def _sc_pos_deg(pos_pad, src, dst, zero16):
  return _make_sc_pos_deg()(pos_pad, src, dst, zero16)


@functools.lru_cache(maxsize=None)
def _make_sc_gather2():
  kern = functools.partial(
      pl.kernel, mesh=_mesh(),
      compiler_params=pltpu.CompilerParams(use_tc_tiling_on_sc=False),
      out_type=(
          jax.ShapeDtypeStruct((_E2, _DP), jnp.float32),
          jax.ShapeDtypeStruct((_E2, _DP), jnp.float32),
      ),
      scratch_types=[
          pltpu.VMEM((_C,), jnp.int32),
          pltpu.VMEM((_C,), jnp.int32),
          pltpu.VMEM((_C,), jnp.int32),
          pltpu.VMEM((_C, _DP), jnp.float32),
          pltpu.VMEM((_C, _DP), jnp.float32),
          pltpu.VMEM((_C, _DP), jnp.float32),
          pltpu.SemaphoreType.DMA,
          pltpu.SemaphoreType.DMA,
          pltpu.SemaphoreType.DMA,
          pltpu.SemaphoreType.DMA,
          pltpu.SemaphoreType.DMA,
          pltpu.SemaphoreType.DMA,
      ],
  )

  @kern
  def body(tab_h, src_h, dst_h, oa_h, ob_h, i0, i1, i2, r0, r1, r2,
           g0, g1, g2, s0, s1, s2):
    c = lax.axis_index("c")
    s = lax.axis_index("s")
    base = (s * _NC + c) * _EPW

    idx = (i0, i1, i2)
    rows = (r0, r1, r2)
    gsem = (g0, g1, g2)
    ssem = (s0, s1, s2)
    nchunk = _EPW // _C
    jobs = [(src_h, oa_h, j * _C) for j in range(nchunk)]
    jobs += [(dst_h, ob_h, j * _C) for j in range(nchunk)]

    gh = [None, None, None]
    sh = [None, None, None]
    for k, (ih, oh, rel) in enumerate(jobs):
      slot = k % 3
      off = pl.multiple_of(base + rel, 8)
      if k >= 3:
        sh[slot].wait()
      pltpu.sync_copy(ih.at[pl.ds(off, _C)], idx[slot])
      gh[slot] = pltpu.async_copy(tab_h.at[idx[slot]], rows[slot], gsem[slot])
      if k >= 1:
        pk, ph, po = k - 1, jobs[k - 1][1], jobs[k - 1][2]
        t = pk % 3
        gh[t].wait()
        poff = pl.multiple_of(base + po, 8)
        sh[t] = pltpu.make_async_copy(rows[t], ph.at[pl.ds(poff, _C)],
                                      ssem[t])
        sh[t].start()
    last = len(jobs) - 1
    t = last % 3
    gh[t].wait()
    loff = pl.multiple_of(base + jobs[last][2], 8)
    sh[t] = pltpu.make_async_copy(rows[t], jobs[last][1].at[pl.ds(loff, _C)],
                                  ssem[t])
    sh[t].start()
    for t in range(3):
      sh[t].wait()

  return body


def _sc_gather2(tab, src, dst):
  return _make_sc_gather2()(tab, src, dst)


@functools.lru_cache(maxsize=None)
def _make_sc_scatter():
  kern = functools.partial(
      pl.kernel, mesh=_mesh(),
      compiler_params=pltpu.CompilerParams(use_tc_tiling_on_sc=False),
      out_type=jax.ShapeDtypeStruct((2 * _NP, _DP), jnp.float32),
      scratch_types=[
          pltpu.VMEM((_C,), jnp.int32),
          pltpu.VMEM((_C, _DP), jnp.float32),
          pltpu.VMEM_SHARED((_NP, _DP), jnp.float32),
          pltpu.SemaphoreType.DMA,
      ],
  )

  @kern
  def body(msg_h, dst_h, zero_h, out_h, idx_v, rows_v, acc_sh, sem):
    c = lax.axis_index("c")
    s = lax.axis_index("s")
    base = (s * _NC + c) * _EPW

    pltpu.sync_copy(zero_h, acc_sh.at[pl.ds(s * _RPT, _RPT)])
    plsc.subcore_barrier()

    def step(j, carry):
      off = pl.multiple_of(base + j * _C, 8)
      pltpu.sync_copy(dst_h.at[pl.ds(off, _C)], idx_v)
      pltpu.sync_copy(msg_h.at[pl.ds(off, _C)], rows_v)
      pltpu.sync_copy(rows_v, acc_sh.at[idx_v], add=True)
      return carry
    lax.fori_loop(0, _EPW // _C, step, 0)

    plsc.subcore_barrier()
    out_off = pl.multiple_of(c * _NP + s * _RPT, 8)
    pltpu.sync_copy(acc_sh.at[pl.ds(s * _RPT, _RPT)],
                    out_h.at[pl.ds(out_off, _RPT)])

  return body


def _sc_scatter(msg, dst, zero32):
  return _make_sc_scatter()(msg, dst, zero32)


def _nemb_body(x_ref, w1_ref, b1_ref, w2_ref, b2_ref, h_ref):
    a = jnp.maximum(
        jnp.dot(x_ref[...], w1_ref[...], preferred_element_type=jnp.float32)
        + b1_ref[...], 0.0)
    h_ref[...] = jnp.dot(a, w2_ref[...],
                         preferred_element_type=jnp.float32) + b2_ref[...]


def _efeat_body(ps_ref, pd_ref, ea_ref, w1a_ref, w1b_ref, b1_ref,
                w2_ref, b2_ref, off_ref, aa_ref, ab_ref, ac_ref, c0_ref,
                g4_ref, e4_ref, es_ref, sh_ref, *, coeff):
    f32 = jnp.float32
    vec4 = pd_ref[...] - ps_ref[...]
    d2 = jnp.dot(vec4 * vec4, g4_ref[...],
                 preferred_element_type=f32)
    d = jnp.sqrt(d2 + 1e-12)
    d4e = jnp.dot(d, e4_ref[...], preferred_element_type=f32)
    r4e = jnp.dot(1.0 / d, e4_ref[...], preferred_element_type=f32)
    dist4 = jnp.exp(coeff * (d4e - off_ref[...]) ** 2)
    e1 = jnp.maximum(
        jnp.dot(ea_ref[...], w1a_ref[...], preferred_element_type=f32)
        + jnp.dot(dist4, w1b_ref[...], preferred_element_type=f32)
        + b1_ref[...], 0.0)
    es_ref[...] = jnp.dot(e1, w2_ref[...],
                          preferred_element_type=f32) + b2_ref[...]
    vhat4 = vec4 * r4e
    ga = jnp.dot(vhat4, aa_ref[...], preferred_element_type=f32)
    gb = jnp.dot(vhat4, ab_ref[...], preferred_element_type=f32)
    gc = jnp.dot(vhat4, ac_ref[...], preferred_element_type=f32)
    sh_ref[...] = ga * gb + gc + c0_ref[...]


def _dense_body(es_ref, hs_ref, hd_ref, sh_ref, w1_ref, b1_ref,
                w2_ref, b2_ref, bm_ref, fm_ref, out_ref):
    hs4 = hs_ref[...]
    z4 = jnp.concatenate([es_ref[...], hs4, hd_ref[...]], axis=1)
    hid4 = jnp.maximum(
        jnp.dot(z4, w1_ref[...], preferred_element_type=jnp.float32)
        + b1_ref[...], 0.0)
    t4 = jnp.dot(hid4, w2_ref[...],
                 preferred_element_type=jnp.float32) + b2_ref[...]
    s4 = jnp.dot(sh_ref[...], bm_ref[...],
                 preferred_element_type=jnp.float32)
    msg4 = jnp.dot(s4 * t4, fm_ref[...],
                   preferred_element_type=jnp.float32)
    out_ref[...] = msg4 * hs4


def _upd_body(h_ref, p0_ref, p1_ref, g0_ref, g1_ref, ho_ref):
    deg = g0_ref[0, :, 0:1] + g1_ref[0, :, 0:1]
    agg = (p0_ref[0] + p1_ref[0]) / jnp.maximum(deg, 1.0)
    ho_ref[...] = h_ref[...] + agg


def _final_body(h_ref, w_ref, b_ref, o_ref):
    o_ref[...] = jnp.dot(h_ref[...], w_ref[...],
                         preferred_element_type=jnp.float32) + b_ref[...]


def _wspec(shape):
    nd = len(shape)
    return pl.BlockSpec(shape, lambda i: (0,) * nd)


def kernel(x, pos, edge_index, edge_attr, Wn1, bn1, Wn2, bn2, We1, be1,
           We2, be2, Wm1, bm1, Wm2, bm2, Wout, bout):
    f32 = jnp.float32
    src = jnp.pad(edge_index[0], (0, _E2 - _E), constant_values=_N)
    dst = jnp.pad(edge_index[1], (0, _E2 - _E), constant_values=_N)
    ea4 = jnp.pad(edge_attr.reshape(_E // 4, 16),
                  ((0, (_E2 - _E) // 4), (0, 0)))

    eye4 = np.eye(4, dtype=np.float32)
    pos_pad = jnp.zeros((_NP, _DP), f32).at[:_N, :3].set(pos)
    Wn2p = jnp.pad(Wn2, ((0, 0), (0, _DP - _NS)))
    bn2p = jnp.pad(bn2, (0, _DP - _NS))
    We1a4 = jnp.kron(eye4, jnp.pad(We1[:4], ((0, 0), (0, 8))))
    We1b4 = jnp.kron(eye4, jnp.pad(We1[4:], ((0, 0), (0, 8))))
    be1_4 = jnp.tile(jnp.pad(be1, (0, 8)), 4)
    We2p4 = jnp.kron(eye4, jnp.pad(We2, ((0, 8), (0, 8))))
    be2p_4 = jnp.tile(jnp.pad(be2, (0, 8)), 4)
    W1blk = jnp.zeros((4, 384, 288), f32)
    for slot in range(4):
        r = slot * 32
        cidx = slot * 72
        W1blk = W1blk.at[:, r:r + 24, cidx:cidx + 72].set(Wm1[:, 0:24])
        W1blk = W1blk.at[:, 128 + r:128 + r + 24, cidx:cidx + 72].set(
            Wm1[:, 24:48])
        W1blk = W1blk.at[:, 256 + r:256 + r + 24, cidx:cidx + 72].set(
            Wm1[:, 48:72])
    b1_4 = jnp.tile(bm1, (1, 4))
    W2p = jnp.pad(Wm2.reshape(4, 72, 9, _NS),
                  ((0, 0), (0, 0), (0, 0), (0, _DP - _NS))).reshape(4, 72, 288)
    b2p = jnp.pad(bm2.reshape(4, 9, _NS),
                  ((0, 0), (0, 0), (0, _DP - _NS))).reshape(4, 288)
    W2p4 = jnp.stack([jnp.kron(eye4, W2p[i]) for i in range(4)])
    b2_4 = jnp.tile(b2p, (1, 4))
    Woutp = jnp.pad(Wout, ((0, _DP - _NS), (0, 0)))
    lanes = np.arange(288)
    Bm = (lanes[None, :] // _DP == np.arange(16)[:, None]).astype(np.float32)
    Fm = (lanes[:, None] % _DP == np.arange(_DP)[None, :]).astype(np.float32)
    Bm4 = jnp.asarray(np.kron(eye4, np.pad(Bm, ((0, 16), (0, 0)))))
    F4 = jnp.asarray(np.kron(eye4, Fm))
    G4 = jnp.asarray(np.kron(eye4, np.ones((32, 1), np.float32)))
    E4 = jnp.asarray(np.kron(eye4, np.ones((1, 32), np.float32)))

    offs = np.linspace(0.0, 5.0, _DP, dtype=np.float32)
    coeff = float(-0.5 / (float(offs[1]) - float(offs[0])) ** 2)
    off4 = jnp.asarray(np.tile(offs, 4))[None, :]

    c1 = float(np.sqrt(3.0))
    c2 = float(np.sqrt(15.0))
    c3 = float(np.sqrt(5.0) / 2.0)
    Aa = np.zeros((16, 16), np.float32)
    Ab = np.zeros((16, 16), np.float32)
    Ac = np.zeros((16, 16), np.float32)
    c0 = np.zeros((1, 16), np.float32)
    X, Y, Z = 0, 1, 2
    c0[0, 0] = 1.0
    Ac[X, 1] = c1
    Ac[Y, 2] = c1
    Ac[Z, 3] = c1
    Aa[X, 4] = 1.0
    Ab[Y, 4] = c2
    Aa[Y, 5] = 1.0
    Ab[Z, 5] = c2
    Aa[Z, 6] = 1.0
    Ab[Z, 6] = 3.0 * c3
    c0[0, 6] = -c3
    Aa[X, 7] = 1.0
    Ab[Z, 7] = c2
    Aa[X, 8] = 1.0
    Aa[Y, 8] = 1.0
    Ab[X, 8] = c2 / 2.0
    Ab[Y, 8] = -c2 / 2.0
    def _blk32(m):
        return jnp.asarray(np.kron(eye4, np.pad(m, ((0, 16), (0, 16)))))
    Aa4, Ab4, Ac4 = _blk32(Aa), _blk32(Ab), _blk32(Ac)
    c0_4 = jnp.asarray(np.tile(np.pad(c0, ((0, 0), (0, 16))), (1, 4)))

    zero16 = jnp.zeros((_RPT, _PW), f32)
    zero32 = jnp.zeros((_RPT, _DP), f32)

    h = pl.pallas_call(
        _nemb_body,
        grid=(_N // 1000,),
        in_specs=[pl.BlockSpec((1000, 128), lambda i: (i, 0)),
                  _wspec((128, _NS)), _wspec((1, _NS)),
                  _wspec((_NS, _DP)), _wspec((1, _DP))],
        out_specs=pl.BlockSpec((1000, _DP), lambda i: (i, 0)),
        out_shape=jax.ShapeDtypeStruct((_N, _DP), f32),
    )(x, Wn1, bn1[None], Wn2p, bn2p[None])

    ps, pd, degp = _sc_pos_deg(pos_pad, src, dst, zero16)
    degp = degp.reshape(2, _NP, _PW)
    ps4 = ps.reshape(_E2 // 4, 128)
    pd4 = pd.reshape(_E2 // 4, 128)

    es4, sh4 = pl.pallas_call(
        functools.partial(_efeat_body, coeff=coeff),
        grid=(_E2 // _BE,),
        in_specs=[pl.BlockSpec((_BE // 4, 128), lambda i: (i, 0)),
                  pl.BlockSpec((_BE // 4, 128), lambda i: (i, 0)),
                  pl.BlockSpec((_BE // 4, 16), lambda i: (i, 0)),
                  _wspec((16, 128)), _wspec((128, 128)), _wspec((1, 128)),
                  _wspec((128, 128)), _wspec((1, 128)), _wspec((1, 128)),
                  _wspec((128, 128)), _wspec((128, 128)), _wspec((128, 128)),
                  _wspec((1, 128)), _wspec((128, 4)), _wspec((4, 128))],
        out_specs=(pl.BlockSpec((_BE // 4, 128), lambda i: (i, 0)),
                   pl.BlockSpec((_BE // 4, 128), lambda i: (i, 0))),
        out_shape=(jax.ShapeDtypeStruct((_E2 // 4, 128), f32),
                   jax.ShapeDtypeStruct((_E2 // 4, 128), f32)),
    )(ps4, pd4, ea4, We1a4, We1b4, be1_4[None], We2p4, be2p_4[None], off4,
      Aa4, Ab4, Ac4, c0_4, G4, E4)

    hp = jnp.zeros((_NP, _DP), f32).at[:_N].set(h)
    for i in range(4):
        hs, hd = _sc_gather2(hp, src, dst)
        hs4 = hs.reshape(_E2 // 4, 128)
        hd4 = hd.reshape(_E2 // 4, 128)
        msg4 = pl.pallas_call(
            _dense_body,
            grid=(_E2 // _BE,),
            in_specs=[pl.BlockSpec((_BE // 4, 128), lambda j: (j, 0)),
                      pl.BlockSpec((_BE // 4, 128), lambda j: (j, 0)),
                      pl.BlockSpec((_BE // 4, 128), lambda j: (j, 0)),
                      pl.BlockSpec((_BE // 4, 128), lambda j: (j, 0)),
                      _wspec((384, 288)), _wspec((1, 288)),
                      _wspec((288, 1152)), _wspec((1, 1152)),
                      _wspec((128, 1152)), _wspec((1152, 128))],
            out_specs=pl.BlockSpec((_BE // 4, 128), lambda j: (j, 0)),
            out_shape=jax.ShapeDtypeStruct((_E2 // 4, 128), f32),
        )(es4, hs4, hd4, sh4, W1blk[i], b1_4[i][None],
          W2p4[i], b2_4[i][None], Bm4, F4)
        msg = msg4.reshape(_E2, _DP)

        parts = _sc_scatter(msg, dst, zero32).reshape(2, _NP, _DP)

        h = pl.pallas_call(
            _upd_body,
            grid=(_N // _BN,),
            in_specs=[pl.BlockSpec((_BN, _DP), lambda j: (j, 0)),
                      pl.BlockSpec((1, _BN, _DP), lambda j: (0, j, 0)),
                      pl.BlockSpec((1, _BN, _DP), lambda j: (1, j, 0)),
                      pl.BlockSpec((1, _BN, _PW), lambda j: (0, j, 0)),
                      pl.BlockSpec((1, _BN, _PW), lambda j: (1, j, 0))],
            out_specs=pl.BlockSpec((_BN, _DP), lambda j: (j, 0)),
            out_shape=jax.ShapeDtypeStruct((_N, _DP), f32),
        )(h, parts, parts, degp, degp)
        if i < 3:
            hp = jnp.zeros((_NP, _DP), f32).at[:_N].set(h)

    return pl.pallas_call(
        _final_body,
        grid=(_N // _BN,),
        in_specs=[pl.BlockSpec((_BN, _DP), lambda j: (j, 0)),
                  _wspec((_DP, _NS)), _wspec((1, _NS))],
        out_specs=pl.BlockSpec((_BN, _NS), lambda j: (j, 0)),
        out_shape=jax.ShapeDtypeStruct((_N, _NS), f32),
    )(h, Woutp, bout[None])


# speedup vs baseline: 4.1418x; 1.0782x over previous
import functools

import jax
import jax.numpy as jnp
import numpy as np
from jax import lax
from jax.experimental import pallas as pl
from jax.experimental.pallas import tpu as pltpu
from jax.experimental.pallas import tpu_sc as plsc

_N = 10000
_E = 160000
_E2 = 163840
_NS = 24
_DP = 32
_PW = 16
_NC = 2
_NT = 16
_NW = _NC * _NT
_EPW = _E2 // _NW
_C = 1024
_CH = 512
_NP = 10240
_RPT = _NP // _NT

_BE = 2048
_BN = 2000


def _mesh():
    return plsc.VectorSubcoreMesh(core_axis_name="c", subcore_axis_name="s")


@functools.lru_cache(maxsize=None)
def _make_sc_pos_deg():
  kern = functools.partial(
      pl.kernel, mesh=_mesh(),
      compiler_params=pltpu.CompilerParams(use_tc_tiling_on_sc=False),
      out_type=(
          jax.ShapeDtypeStruct((_E2, _DP), jnp.float32),
          jax.ShapeDtypeStruct((_E2, _DP), jnp.float32),
          jax.ShapeDtypeStruct((2 * _NP, _PW), jnp.float32),
      ),
      scratch_types=[
          pltpu.VMEM((_C,), jnp.int32),
          pltpu.VMEM((_C, _DP), jnp.float32),
          pltpu.VMEM((_C, _PW), jnp.float32),
          pltpu.VMEM_SHARED((_NP, _PW), jnp.float32),
          pltpu.SemaphoreType.DMA,
      ],
  )

  @kern
  def body(pos_h, src_h, dst_h, zero_h, ps_h, pd_h, deg_h,
           idx_v, rows_v, ones_v, acc_sh, sem):
    c = lax.axis_index("c")
    s = lax.axis_index("s")
    base = (s * _NC + c) * _EPW

    def fill_ones(i, carry):
      ones_v[i] = jnp.ones((_PW,), jnp.float32)
      return carry
    lax.fori_loop(0, _C, fill_ones, 0)
    pltpu.sync_copy(zero_h, acc_sh.at[pl.ds(s * _RPT, _RPT)])
    plsc.subcore_barrier()

    def step_src(j, carry):
      off = pl.multiple_of(base + j * _C, 8)
      pltpu.sync_copy(src_h.at[pl.ds(off, _C)], idx_v)
      pltpu.async_copy(pos_h.at[idx_v], rows_v, sem).wait()
      pltpu.sync_copy(rows_v, ps_h.at[pl.ds(off, _C)])
      return carry
    lax.fori_loop(0, _EPW // _C, step_src, 0)

    def step_dst(j, carry):
      off = pl.multiple_of(base + j * _C, 8)
      pltpu.sync_copy(dst_h.at[pl.ds(off, _C)], idx_v)
      pltpu.async_copy(pos_h.at[idx_v], rows_v, sem).wait()
      pltpu.sync_copy(rows_v, pd_h.at[pl.ds(off, _C)])
      pltpu.sync_copy(ones_v, acc_sh.at[idx_v], add=True)
      return carry
    lax.fori_loop(0, _EPW // _C, step_dst, 0)

    plsc.subcore_barrier()
    out_off = pl.multiple_of(c * _NP + s * _RPT, 8)
    pltpu.sync_copy(acc_sh.at[pl.ds(s * _RPT, _RPT)],
                    deg_h.at[pl.ds(out_off, _RPT)])

  return body


def _sc_pos_deg(pos_pad, src, dst, zero16):
  return _make_sc_pos_deg()(pos_pad, src, dst, zero16)


@functools.lru_cache(maxsize=None)
def _make_sc_gather2(half):
  kern = functools.partial(
      pl.kernel, mesh=_mesh(),
      compiler_params=pltpu.CompilerParams(use_tc_tiling_on_sc=False),
      out_type=(
          jax.ShapeDtypeStruct((_E2 // 2, _DP), jnp.float32),
          jax.ShapeDtypeStruct((_E2 // 2, _DP), jnp.float32),
      ),
      scratch_types=[
          pltpu.VMEM((_CH,), jnp.int32),
          pltpu.VMEM((_CH,), jnp.int32),
          pltpu.VMEM((_CH,), jnp.int32),
          pltpu.VMEM((_CH, _DP), jnp.float32),
          pltpu.VMEM((_CH, _DP), jnp.float32),
          pltpu.VMEM((_CH, _DP), jnp.float32),
          pltpu.SemaphoreType.DMA,
          pltpu.SemaphoreType.DMA,
          pltpu.SemaphoreType.DMA,
          pltpu.SemaphoreType.DMA,
          pltpu.SemaphoreType.DMA,
          pltpu.SemaphoreType.DMA,
      ],
  )

  @kern
  def body(tab_h, src_h, dst_h, oa_h, ob_h, i0, i1, i2, r0, r1, r2,
           g0, g1, g2, s0, s1, s2):
    c = lax.axis_index("c")
    s = lax.axis_index("s")
    hw = _EPW // 2
    base = half * (_E2 // 2) + (s * _NC + c) * hw
    obase = (s * _NC + c) * hw

    idx = (i0, i1, i2)
    rows = (r0, r1, r2)
    gsem = (g0, g1, g2)
    ssem = (s0, s1, s2)
    nchunk = hw // _CH
    jobs = [(src_h, oa_h, j * _CH) for j in range(nchunk)]
    jobs += [(dst_h, ob_h, j * _CH) for j in range(nchunk)]

    gh = [None, None, None]
    sh = [None, None, None]
    for k, (ih, oh, rel) in enumerate(jobs):
      slot = k % 3
      off = pl.multiple_of(base + rel, 8)
      if k >= 3:
        sh[slot].wait()
      pltpu.sync_copy(ih.at[pl.ds(off, _CH)], idx[slot])
      gh[slot] = pltpu.async_copy(tab_h.at[idx[slot]], rows[slot], gsem[slot])
      if k >= 1:
        pk, ph, po = k - 1, jobs[k - 1][1], jobs[k - 1][2]
        t = pk % 3
        gh[t].wait()
        poff = pl.multiple_of(obase + po, 8)
        sh[t] = pltpu.make_async_copy(rows[t], ph.at[pl.ds(poff, _CH)],
                                      ssem[t])
        sh[t].start()
    last = len(jobs) - 1
    t = last % 3
    gh[t].wait()
    loff = pl.multiple_of(obase + jobs[last][2], 8)
    sh[t] = pltpu.make_async_copy(rows[t], jobs[last][1].at[pl.ds(loff, _CH)],
                                  ssem[t])
    sh[t].start()
    for t in range(3):
      sh[t].wait()

  return body


def _sc_gather2(tab, src, dst, half):
  return _make_sc_gather2(half)(tab, src, dst)


@functools.lru_cache(maxsize=None)
def _make_sc_scatter(half):
  kern = functools.partial(
      pl.kernel, mesh=_mesh(),
      compiler_params=pltpu.CompilerParams(use_tc_tiling_on_sc=False),
      out_type=jax.ShapeDtypeStruct((2 * _NP, _DP), jnp.float32),
      scratch_types=[
          pltpu.VMEM((_CH,), jnp.int32),
          pltpu.VMEM((_CH, _DP), jnp.float32),
          pltpu.VMEM_SHARED((_NP, _DP), jnp.float32),
          pltpu.SemaphoreType.DMA,
      ],
  )

  @kern
  def body(msg_h, dst_h, zero_h, out_h, idx_v, rows_v, acc_sh, sem):
    c = lax.axis_index("c")
    s = lax.axis_index("s")
    hw = _EPW // 2
    base = half * (_E2 // 2) + (s * _NC + c) * hw
    mbase = (s * _NC + c) * hw

    pltpu.sync_copy(zero_h, acc_sh.at[pl.ds(s * _RPT, _RPT)])
    plsc.subcore_barrier()

    def step(j, carry):
      off = pl.multiple_of(base + j * _CH, 8)
      moff = pl.multiple_of(mbase + j * _CH, 8)
      pltpu.sync_copy(dst_h.at[pl.ds(off, _CH)], idx_v)
      pltpu.sync_copy(msg_h.at[pl.ds(moff, _CH)], rows_v)
      pltpu.sync_copy(rows_v, acc_sh.at[idx_v], add=True)
      return carry
    lax.fori_loop(0, hw // _CH, step, 0)

    plsc.subcore_barrier()
    out_off = pl.multiple_of(c * _NP + s * _RPT, 8)
    pltpu.sync_copy(acc_sh.at[pl.ds(s * _RPT, _RPT)],
                    out_h.at[pl.ds(out_off, _RPT)])

  return body


def _sc_scatter(msg, dst, zero32, half):
  return _make_sc_scatter(half)(msg, dst, zero32)


def _nemb_body(x_ref, w1_ref, b1_ref, w2_ref, b2_ref, h_ref):
    a = jnp.maximum(
        jnp.dot(x_ref[...], w1_ref[...], preferred_element_type=jnp.float32)
        + b1_ref[...], 0.0)
    h_ref[...] = jnp.dot(a, w2_ref[...],
                         preferred_element_type=jnp.float32) + b2_ref[...]


def _efeat_body(ps_ref, pd_ref, ea_ref, w1a_ref, w1b_ref, b1_ref,
                w2_ref, b2_ref, off_ref, aa_ref, ab_ref, ac_ref, c0_ref,
                g4_ref, e4_ref, es_ref, sh_ref, *, coeff):
    f32 = jnp.float32
    vec4 = pd_ref[...] - ps_ref[...]
    d2 = jnp.dot(vec4 * vec4, g4_ref[...],
                 preferred_element_type=f32)
    d = jnp.sqrt(d2 + 1e-12)
    d4e = jnp.dot(d, e4_ref[...], preferred_element_type=f32)
    r4e = jnp.dot(1.0 / d, e4_ref[...], preferred_element_type=f32)
    dist4 = jnp.exp(coeff * (d4e - off_ref[...]) ** 2)
    e1 = jnp.maximum(
        jnp.dot(ea_ref[...], w1a_ref[...], preferred_element_type=f32)
        + jnp.dot(dist4, w1b_ref[...], preferred_element_type=f32)
        + b1_ref[...], 0.0)
    es_ref[...] = jnp.dot(e1, w2_ref[...],
                          preferred_element_type=f32) + b2_ref[...]
    vhat4 = vec4 * r4e
    ga = jnp.dot(vhat4, aa_ref[...], preferred_element_type=f32)
    gb = jnp.dot(vhat4, ab_ref[...], preferred_element_type=f32)
    gc = jnp.dot(vhat4, ac_ref[...], preferred_element_type=f32)
    sh_ref[...] = ga * gb + gc + c0_ref[...]


def _dense_body(es_ref, hs_ref, hd_ref, sh_ref, w1_ref, b1_ref,
                w2_ref, b2_ref, bm_ref, fm_ref, out_ref):
    hs4 = hs_ref[...]
    z4 = jnp.concatenate([es_ref[...], hs4, hd_ref[...]], axis=1)
    hid4 = jnp.maximum(
        jnp.dot(z4, w1_ref[...], preferred_element_type=jnp.float32)
        + b1_ref[...], 0.0)
    t4 = jnp.dot(hid4, w2_ref[...],
                 preferred_element_type=jnp.float32) + b2_ref[...]
    s4 = jnp.dot(sh_ref[...], bm_ref[...],
                 preferred_element_type=jnp.float32)
    msg4 = jnp.dot(s4 * t4, fm_ref[...],
                   preferred_element_type=jnp.float32)
    out_ref[...] = msg4 * hs4


def _upd_body(h_ref, pa0_ref, pa1_ref, pb0_ref, pb1_ref, g0_ref, g1_ref,
              ho_ref):
    deg = g0_ref[0, :, 0:1] + g1_ref[0, :, 0:1]
    agg = (pa0_ref[0] + pa1_ref[0] + pb0_ref[0] + pb1_ref[0]) \
        / jnp.maximum(deg, 1.0)
    ho_ref[...] = h_ref[...] + agg


def _final_body(h_ref, w_ref, b_ref, o_ref):
    o_ref[...] = jnp.dot(h_ref[...], w_ref[...],
                         preferred_element_type=jnp.float32) + b_ref[...]


def _wspec(shape):
    nd = len(shape)
    return pl.BlockSpec(shape, lambda i: (0,) * nd)


def kernel(x, pos, edge_index, edge_attr, Wn1, bn1, Wn2, bn2, We1, be1,
           We2, be2, Wm1, bm1, Wm2, bm2, Wout, bout):
    f32 = jnp.float32
    src = jnp.pad(edge_index[0], (0, _E2 - _E), constant_values=_N)
    dst = jnp.pad(edge_index[1], (0, _E2 - _E), constant_values=_N)
    ea4 = jnp.pad(edge_attr.reshape(_E // 4, 16),
                  ((0, (_E2 - _E) // 4), (0, 0)))

    eye4 = np.eye(4, dtype=np.float32)
    pos_pad = jnp.zeros((_NP, _DP), f32).at[:_N, :3].set(pos)
    Wn2p = jnp.pad(Wn2, ((0, 0), (0, _DP - _NS)))
    bn2p = jnp.pad(bn2, (0, _DP - _NS))
    We1a4 = jnp.kron(eye4, jnp.pad(We1[:4], ((0, 0), (0, 8))))
    We1b4 = jnp.kron(eye4, jnp.pad(We1[4:], ((0, 0), (0, 8))))
    be1_4 = jnp.tile(jnp.pad(be1, (0, 8)), 4)
    We2p4 = jnp.kron(eye4, jnp.pad(We2, ((0, 8), (0, 8))))
    be2p_4 = jnp.tile(jnp.pad(be2, (0, 8)), 4)
    W1blk = jnp.zeros((4, 384, 288), f32)
    for slot in range(4):
        r = slot * 32
        cidx = slot * 72
        W1blk = W1blk.at[:, r:r + 24, cidx:cidx + 72].set(Wm1[:, 0:24])
        W1blk = W1blk.at[:, 128 + r:128 + r + 24, cidx:cidx + 72].set(
            Wm1[:, 24:48])
        W1blk = W1blk.at[:, 256 + r:256 + r + 24, cidx:cidx + 72].set(
            Wm1[:, 48:72])
    b1_4 = jnp.tile(bm1, (1, 4))
    W2p = jnp.pad(Wm2.reshape(4, 72, 9, _NS),
                  ((0, 0), (0, 0), (0, 0), (0, _DP - _NS))).reshape(4, 72, 288)
    b2p = jnp.pad(bm2.reshape(4, 9, _NS),
                  ((0, 0), (0, 0), (0, _DP - _NS))).reshape(4, 288)
    W2p4 = jnp.stack([jnp.kron(eye4, W2p[i]) for i in range(4)])
    b2_4 = jnp.tile(b2p, (1, 4))
    Woutp = jnp.pad(Wout, ((0, _DP - _NS), (0, 0)))
    lanes = np.arange(288)
    Bm = (lanes[None, :] // _DP == np.arange(16)[:, None]).astype(np.float32)
    Fm = (lanes[:, None] % _DP == np.arange(_DP)[None, :]).astype(np.float32)
    Bm4 = jnp.asarray(np.kron(eye4, np.pad(Bm, ((0, 16), (0, 0)))))
    F4 = jnp.asarray(np.kron(eye4, Fm))
    G4 = jnp.asarray(np.kron(eye4, np.ones((32, 1), np.float32)))
    E4 = jnp.asarray(np.kron(eye4, np.ones((1, 32), np.float32)))

    offs = np.linspace(0.0, 5.0, _DP, dtype=np.float32)
    coeff = float(-0.5 / (float(offs[1]) - float(offs[0])) ** 2)
    off4 = jnp.asarray(np.tile(offs, 4))[None, :]

    c1 = float(np.sqrt(3.0))
    c2 = float(np.sqrt(15.0))
    c3 = float(np.sqrt(5.0) / 2.0)
    Aa = np.zeros((16, 16), np.float32)
    Ab = np.zeros((16, 16), np.float32)
    Ac = np.zeros((16, 16), np.float32)
    c0 = np.zeros((1, 16), np.float32)
    X, Y, Z = 0, 1, 2
    c0[0, 0] = 1.0
    Ac[X, 1] = c1
    Ac[Y, 2] = c1
    Ac[Z, 3] = c1
    Aa[X, 4] = 1.0
    Ab[Y, 4] = c2
    Aa[Y, 5] = 1.0
    Ab[Z, 5] = c2
    Aa[Z, 6] = 1.0
    Ab[Z, 6] = 3.0 * c3
    c0[0, 6] = -c3
    Aa[X, 7] = 1.0
    Ab[Z, 7] = c2
    Aa[X, 8] = 1.0
    Aa[Y, 8] = 1.0
    Ab[X, 8] = c2 / 2.0
    Ab[Y, 8] = -c2 / 2.0
    def _blk32(m):
        return jnp.asarray(np.kron(eye4, np.pad(m, ((0, 16), (0, 16)))))
    Aa4, Ab4, Ac4 = _blk32(Aa), _blk32(Ab), _blk32(Ac)
    c0_4 = jnp.asarray(np.tile(np.pad(c0, ((0, 0), (0, 16))), (1, 4)))

    zero16 = jnp.zeros((_RPT, _PW), f32)
    zero32 = jnp.zeros((_RPT, _DP), f32)

    h = pl.pallas_call(
        _nemb_body,
        grid=(_N // 1000,),
        in_specs=[pl.BlockSpec((1000, 128), lambda i: (i, 0)),
                  _wspec((128, _NS)), _wspec((1, _NS)),
                  _wspec((_NS, _DP)), _wspec((1, _DP))],
        out_specs=pl.BlockSpec((1000, _DP), lambda i: (i, 0)),
        out_shape=jax.ShapeDtypeStruct((_N, _DP), f32),
    )(x, Wn1, bn1[None], Wn2p, bn2p[None])

    ps, pd, degp = _sc_pos_deg(pos_pad, src, dst, zero16)
    degp = degp.reshape(2, _NP, _PW)
    ps4 = ps.reshape(_E2 // 4, 128)
    pd4 = pd.reshape(_E2 // 4, 128)

    es4, sh4 = pl.pallas_call(
        functools.partial(_efeat_body, coeff=coeff),
        grid=(_E2 // _BE,),
        in_specs=[pl.BlockSpec((_BE // 4, 128), lambda i: (i, 0)),
                  pl.BlockSpec((_BE // 4, 128), lambda i: (i, 0)),
                  pl.BlockSpec((_BE // 4, 16), lambda i: (i, 0)),
                  _wspec((16, 128)), _wspec((128, 128)), _wspec((1, 128)),
                  _wspec((128, 128)), _wspec((1, 128)), _wspec((1, 128)),
                  _wspec((128, 128)), _wspec((128, 128)), _wspec((128, 128)),
                  _wspec((1, 128)), _wspec((128, 4)), _wspec((4, 128))],
        out_specs=(pl.BlockSpec((_BE // 4, 128), lambda i: (i, 0)),
                   pl.BlockSpec((_BE // 4, 128), lambda i: (i, 0))),
        out_shape=(jax.ShapeDtypeStruct((_E2 // 4, 128), f32),
                   jax.ShapeDtypeStruct((_E2 // 4, 128), f32)),
    )(ps4, pd4, ea4, We1a4, We1b4, be1_4[None], We2p4, be2p_4[None], off4,
      Aa4, Ab4, Ac4, c0_4, G4, E4)

    nblk_h = _E2 // 2 // _BE

    def dense_half(hs, hd, i, half):
        hs4 = hs.reshape(_E2 // 8, 128)
        hd4 = hd.reshape(_E2 // 8, 128)
        emap = lambda j, hh=half: (j + hh * nblk_h, 0)
        hmap = lambda j: (j, 0)
        return pl.pallas_call(
            _dense_body,
            grid=(nblk_h,),
            in_specs=[pl.BlockSpec((_BE // 4, 128), emap),
                      pl.BlockSpec((_BE // 4, 128), hmap),
                      pl.BlockSpec((_BE // 4, 128), hmap),
                      pl.BlockSpec((_BE // 4, 128), emap),
                      _wspec((384, 288)), _wspec((1, 288)),
                      _wspec((288, 1152)), _wspec((1, 1152)),
                      _wspec((128, 1152)), _wspec((1152, 128))],
            out_specs=pl.BlockSpec((_BE // 4, 128), hmap),
            out_shape=jax.ShapeDtypeStruct((_E2 // 8, 128), f32),
        )(es4, hs4, hd4, sh4, W1blk[i], b1_4[i][None],
          W2p4[i], b2_4[i][None], Bm4, F4)

    hp = jnp.zeros((_NP, _DP), f32).at[:_N].set(h)
    for i in range(4):
        hs_a, hd_a = _sc_gather2(hp, src, dst, 0)
        hs_b, hd_b = _sc_gather2(hp, src, dst, 1)
        msg_a = dense_half(hs_a, hd_a, i, 0).reshape(_E2 // 2, _DP)
        parts_a = _sc_scatter(msg_a, dst, zero32, 0).reshape(2, _NP, _DP)
        msg_b = dense_half(hs_b, hd_b, i, 1).reshape(_E2 // 2, _DP)
        parts_b = _sc_scatter(msg_b, dst, zero32, 1).reshape(2, _NP, _DP)

        h = pl.pallas_call(
            _upd_body,
            grid=(_N // _BN,),
            in_specs=[pl.BlockSpec((_BN, _DP), lambda j: (j, 0)),
                      pl.BlockSpec((1, _BN, _DP), lambda j: (0, j, 0)),
                      pl.BlockSpec((1, _BN, _DP), lambda j: (1, j, 0)),
                      pl.BlockSpec((1, _BN, _DP), lambda j: (0, j, 0)),
                      pl.BlockSpec((1, _BN, _DP), lambda j: (1, j, 0)),
                      pl.BlockSpec((1, _BN, _PW), lambda j: (0, j, 0)),
                      pl.BlockSpec((1, _BN, _PW), lambda j: (1, j, 0))],
            out_specs=pl.BlockSpec((_BN, _DP), lambda j: (j, 0)),
            out_shape=jax.ShapeDtypeStruct((_N, _DP), f32),
        )(h, parts_a, parts_a, parts_b, parts_b, degp, degp)
        if i < 3:
            hp = jnp.zeros((_NP, _DP), f32).at[:_N].set(h)

    return pl.pallas_call(
        _final_body,
        grid=(_N // _BN,),
        in_specs=[pl.BlockSpec((_BN, _DP), lambda j: (j, 0)),
                  _wspec((_DP, _NS)), _wspec((1, _NS))],
        out_specs=pl.BlockSpec((_BN, _NS), lambda j: (j, 0)),
        out_shape=jax.ShapeDtypeStruct((_N, _NS), f32),
    )(h, Woutp, bout[None])
